# baseline (speedup 1.0000x reference)
import dataclasses
import functools

import jax
import jax.numpy as jnp
from jax import lax
from jax.experimental import pallas as pl
from jax.experimental.pallas import tpu as pltpu
from jax.experimental.pallas import tpu_sc as plsc

N_NODES = 10000
N_EDGES = 320000
D = 128
DH = D // 2

NC = 2
NS = 16
LANES = 16

CHUNK = 80
E_PER_S = N_EDGES // NS
G_PER_S = E_PER_S // CHUNK
RB = 200
NRB = N_NODES // RB


def _sc_segment_sum(x_l, x_r, src2, dst2):
    mesh = plsc.VectorSubcoreMesh(core_axis_name="c", subcore_axis_name="s")
    cp = pltpu.CompilerParams()
    if "needs_layout_passes" in pltpu.CompilerParams.__dataclass_fields__:
        cp = dataclasses.replace(cp, needs_layout_passes=False)
    if "use_tc_tiling_on_sc" in pltpu.CompilerParams.__dataclass_fields__:
        cp = dataclasses.replace(cp, use_tc_tiling_on_sc=False)

    @functools.partial(
        pl.kernel,
        compiler_params=cp,
        out_type=(
            jax.ShapeDtypeStruct((NC, N_NODES, DH), jnp.float32),
            jax.ShapeDtypeStruct((NS, N_NODES), jnp.float32),
        ),
        mesh=mesh,
        scratch_types=[
            pltpu.VMEM((G_PER_S, CHUNK), jnp.int32),
            pltpu.VMEM((G_PER_S, CHUNK), jnp.int32),
            pltpu.VMEM((CHUNK, DH), jnp.float32),
            pltpu.VMEM((CHUNK, DH), jnp.float32),
            pltpu.VMEM((N_NODES,), jnp.float32),
            pltpu.VMEM((RB, DH), jnp.float32),
            pltpu.VMEM_SHARED((N_NODES, DH), jnp.float32),
            pltpu.SemaphoreType.DMA,
            pltpu.SemaphoreType.DMA,
            pltpu.SemaphoreType.DMA,
            pltpu.SemaphoreType.DMA,
        ],
    )
    def k(xl_hbm, xr_hbm, src_hbm, dst_hbm, s_out, deg_out, src_v, dst_v,
          rows0_v, rows1_v, deg_v, zb_v, s_sh, sem0, sem1, ssem0, ssem1):
        cid = lax.axis_index("c")
        sid = lax.axis_index("s")

        zeros16 = jnp.zeros((LANES,), jnp.float32)
        ones16 = jnp.full((LANES,), 1.0, jnp.float32)

        @pl.loop(0, RB)
        def _(i):
            for j in range(DH // LANES):
                zb_v[i, pl.ds(j * LANES, LANES)] = zeros16

        @pl.loop(0, N_NODES // LANES)
        def _(i):
            deg_v[pl.ds(i * LANES, LANES)] = zeros16

        for t in range(pl.cdiv(NRB, NS)):
            blk = sid + NS * t
            @pl.when(blk < NRB)
            def _():
                pltpu.sync_copy(zb_v, s_sh.at[pl.ds(blk * RB, RB)])

        pltpu.sync_copy(src_hbm.at[sid], src_v)
        pltpu.sync_copy(dst_hbm.at[sid], dst_v)

        plsc.subcore_barrier()

        def deg_update(g):
            for j in range(CHUNK // LANES):
                idx = dst_v[g, pl.ds(j * LANES, LANES)]
                plsc.addupdate_scatter(deg_v, [idx], ones16)

        def main_loop(xh_hbm, do_deg):
            pltpu.async_copy(xh_hbm.at[src_v.at[0]], rows0_v, sem0)
            pltpu.async_copy(xh_hbm.at[src_v.at[1]], rows1_v, sem1)

            @pl.loop(0, G_PER_S, step=2)
            def _(g):
                pltpu.make_async_copy(
                    xh_hbm.at[src_v.at[g]], rows0_v, sem0).wait()
                sc0 = pltpu.async_copy(
                    rows0_v, s_sh.at[dst_v.at[g]], ssem0, add=True)
                if do_deg:
                    deg_update(g)
                pltpu.make_async_copy(
                    xh_hbm.at[src_v.at[g + 1]], rows1_v, sem1).wait()
                sc1 = pltpu.async_copy(
                    rows1_v, s_sh.at[dst_v.at[g + 1]], ssem1, add=True)
                if do_deg:
                    deg_update(g + 1)
                sc0.wait()

                @pl.when(g + 2 < G_PER_S)
                def _():
                    pltpu.async_copy(
                        xh_hbm.at[src_v.at[g + 2]], rows0_v, sem0)

                sc1.wait()

                @pl.when(g + 3 < G_PER_S)
                def _():
                    pltpu.async_copy(
                        xh_hbm.at[src_v.at[g + 3]], rows1_v, sem1)

        @pl.when(cid == 0)
        def _():
            main_loop(xl_hbm, True)

        @pl.when(cid == 1)
        def _():
            main_loop(xr_hbm, False)

        plsc.subcore_barrier()

        for t in range(pl.cdiv(NRB, NS)):
            blk = sid + NS * t
            @pl.when(blk < NRB)
            def _():
                pltpu.sync_copy(s_sh.at[pl.ds(blk * RB, RB)],
                                s_out.at[cid].at[pl.ds(blk * RB, RB)])

        @pl.when(cid == 0)
        def _():
            pltpu.sync_copy(deg_v, deg_out.at[sid])

    return k(x_l, x_r, src2, dst2)


def _tc_body(x_ref, sp_ref, degp_ref, w1_ref, w2a_ref, w2b_ref, b_ref,
             out_ref):
    deg = jnp.sum(degp_ref[...], axis=0)
    r = 1.0 / jnp.maximum(deg, 1.0)
    acc = jnp.dot(x_ref[...], w1_ref[...], preferred_element_type=jnp.float32)
    sw = jnp.dot(sp_ref[0], w2a_ref[...], preferred_element_type=jnp.float32)
    sw += jnp.dot(sp_ref[1], w2b_ref[...], preferred_element_type=jnp.float32)
    out_ref[...] = acc + sw * r[:, None] + b_ref[...]


def _tc_linear(x, s_half, deg_partial, w1, w2a, w2b, b2):
    return pl.pallas_call(
        _tc_body,
        out_shape=jax.ShapeDtypeStruct((N_NODES, D), jnp.float32),
    )(x, s_half, deg_partial, w1, w2a, w2b, b2)


@jax.jit
def kernel(x, edge_index, W, b):
    ei = edge_index.astype(jnp.int32)
    src2 = ei[0].reshape(NS, G_PER_S, CHUNK)
    dst2 = ei[1].reshape(NS, G_PER_S, CHUNK)
    x_l = x[:, :DH]
    x_r = x[:, DH:]
    s_half, deg_partial = _sc_segment_sum(x_l, x_r, src2, dst2)
    w1 = jnp.transpose(W[:, :D])
    w2 = jnp.transpose(W[:, D:])
    return _tc_linear(x, s_half, deg_partial, w1, w2[:DH], w2[DH:],
                      b.reshape(1, D))


# speedup vs baseline: 1.1929x; 1.1929x over previous
import dataclasses
import functools

import jax
import jax.numpy as jnp
from jax import lax
from jax.experimental import pallas as pl
from jax.experimental.pallas import tpu as pltpu
from jax.experimental.pallas import tpu_sc as plsc

N_NODES = 10000
N_EDGES = 320000
D = 128
DH = D // 2

NC = 2
NS = 16
LANES = 16

CHUNK = 80
E_PER_S = N_EDGES // NS
G_PER_S = E_PER_S // CHUNK
RB = 200
NRB = N_NODES // RB


def _sc_segment_sum(xv, src2, dst2):
    mesh = plsc.VectorSubcoreMesh(core_axis_name="c", subcore_axis_name="s")
    cp = pltpu.CompilerParams()
    if "needs_layout_passes" in pltpu.CompilerParams.__dataclass_fields__:
        cp = dataclasses.replace(cp, needs_layout_passes=False)
    if "use_tc_tiling_on_sc" in pltpu.CompilerParams.__dataclass_fields__:
        cp = dataclasses.replace(cp, use_tc_tiling_on_sc=False)

    @functools.partial(
        pl.kernel,
        compiler_params=cp,
        out_type=(
            jax.ShapeDtypeStruct((NC, N_NODES, DH), jnp.float32),
            jax.ShapeDtypeStruct((NS, N_NODES), jnp.float32),
        ),
        mesh=mesh,
        scratch_types=[
            pltpu.VMEM((G_PER_S, CHUNK), jnp.int32),
            pltpu.VMEM((G_PER_S, CHUNK), jnp.int32),
            pltpu.VMEM((CHUNK, DH), jnp.float32),
            pltpu.VMEM((CHUNK, DH), jnp.float32),
            pltpu.VMEM((N_NODES,), jnp.float32),
            pltpu.VMEM((RB, DH), jnp.float32),
            pltpu.VMEM_SHARED((N_NODES, DH), jnp.float32),
            pltpu.SemaphoreType.DMA,
            pltpu.SemaphoreType.DMA,
        ],
    )
    def k(xv_hbm, src_hbm, dst_hbm, s_out, deg_out, src_v, dst_v,
          rows0_v, rows1_v, deg_v, zb_v, s_sh, sem0, sem1):
        cid = lax.axis_index("c")
        sid = lax.axis_index("s")

        zeros16 = jnp.zeros((LANES,), jnp.float32)
        ones16 = jnp.full((LANES,), 1.0, jnp.float32)

        @pl.loop(0, RB)
        def _(i):
            for j in range(DH // LANES):
                zb_v[i, pl.ds(j * LANES, LANES)] = zeros16

        @pl.loop(0, N_NODES // LANES)
        def _(i):
            deg_v[pl.ds(i * LANES, LANES)] = zeros16

        for t in range(pl.cdiv(NRB, NS)):
            blk = sid + NS * t
            @pl.when(blk < NRB)
            def _():
                pltpu.sync_copy(zb_v, s_sh.at[pl.ds(blk * RB, RB)])

        pltpu.sync_copy(src_hbm.at[sid], src_v)
        pltpu.sync_copy(dst_hbm.at[sid], dst_v)

        plsc.subcore_barrier()

        def xform(g):
            for j in range(CHUNK // LANES):
                sl = pl.ds(j * LANES, LANES)
                src_v[g, sl] = src_v[g, sl] * 2 + cid

        def deg_update(g):
            for j in range(CHUNK // LANES):
                idx = dst_v[g, pl.ds(j * LANES, LANES)]
                plsc.addupdate_scatter(deg_v, [idx], ones16)

        xform(0)
        xform(1)
        pltpu.async_copy(xv_hbm.at[src_v.at[0]], rows0_v, sem0)
        pltpu.async_copy(xv_hbm.at[src_v.at[1]], rows1_v, sem1)

        @pl.loop(0, G_PER_S, step=2)
        def _(g):
            @pl.when(g + 2 < G_PER_S)
            def _():
                xform(g + 2)
            pltpu.make_async_copy(
                xv_hbm.at[src_v.at[g]], rows0_v, sem0).wait()
            pltpu.sync_copy(rows0_v, s_sh.at[dst_v.at[g]], add=True)

            @pl.when(g + 2 < G_PER_S)
            def _():
                pltpu.async_copy(xv_hbm.at[src_v.at[g + 2]], rows0_v, sem0)

            @pl.when(cid == 0)
            def _():
                deg_update(g)

            @pl.when(g + 3 < G_PER_S)
            def _():
                xform(g + 3)
            pltpu.make_async_copy(
                xv_hbm.at[src_v.at[g + 1]], rows1_v, sem1).wait()
            pltpu.sync_copy(rows1_v, s_sh.at[dst_v.at[g + 1]], add=True)

            @pl.when(g + 3 < G_PER_S)
            def _():
                pltpu.async_copy(xv_hbm.at[src_v.at[g + 3]], rows1_v, sem1)

            @pl.when(cid == 0)
            def _():
                deg_update(g + 1)

        plsc.subcore_barrier()

        for t in range(pl.cdiv(NRB, NS)):
            blk = sid + NS * t
            @pl.when(blk < NRB)
            def _():
                pltpu.sync_copy(s_sh.at[pl.ds(blk * RB, RB)],
                                s_out.at[cid].at[pl.ds(blk * RB, RB)])

        @pl.when(cid == 0)
        def _():
            pltpu.sync_copy(deg_v, deg_out.at[sid])

    return k(xv, src2, dst2)


def _tc_body(x_ref, sp_ref, degp_ref, w1_ref, w2a_ref, w2b_ref, b_ref,
             out_ref):
    deg = jnp.sum(degp_ref[...], axis=0)
    r = 1.0 / jnp.maximum(deg, 1.0)
    acc = jnp.dot(x_ref[...], w1_ref[...], preferred_element_type=jnp.float32)
    sw = jnp.dot(sp_ref[0], w2a_ref[...], preferred_element_type=jnp.float32)
    sw += jnp.dot(sp_ref[1], w2b_ref[...], preferred_element_type=jnp.float32)
    out_ref[...] = acc + sw * r[:, None] + b_ref[...]


def _tc_linear(x, s_half, deg_partial, w1, w2a, w2b, b2):
    return pl.pallas_call(
        _tc_body,
        out_shape=jax.ShapeDtypeStruct((N_NODES, D), jnp.float32),
    )(x, s_half, deg_partial, w1, w2a, w2b, b2)


@jax.jit
def kernel(x, edge_index, W, b):
    ei = edge_index.astype(jnp.int32)
    src2 = ei[0].reshape(NS, G_PER_S, CHUNK)
    dst2 = ei[1].reshape(NS, G_PER_S, CHUNK)
    xv = x.reshape(NC * N_NODES, DH)
    s_half, deg_partial = _sc_segment_sum(xv, src2, dst2)
    w1 = jnp.transpose(W[:, :D])
    w2 = jnp.transpose(W[:, D:])
    return _tc_linear(x, s_half, deg_partial, w1, w2[:DH], w2[DH:],
                      b.reshape(1, D))
